# baseline (speedup 1.0000x reference)
import functools

import jax
import jax.numpy as jnp
from jax import lax
from jax.experimental import pallas as pl
from jax.experimental.pallas import tpu as pltpu
from jax.experimental.pallas import tpu_sc as plsc

T, D, I, E, K = 2048, 1024, 512, 8, 2
BM = 256
S = 6144
NBLK = S // BM
NC, NS = 2, 16
NW = NC * NS
CH = 32
NCHUNK = T // (NW * CH)


def _dot_t(a, b):
    return lax.dot_general(a, b, (((1,), (1,)), ((), ())))


def _dot_t_bf16(a, b):
    return lax.dot_general(a.astype(jnp.bfloat16), b.astype(jnp.bfloat16),
                           (((1,), (1,)), ((), ())),
                           preferred_element_type=jnp.float32)


def _silu(x):
    return x / (1.0 + jnp.exp(-x))


def _router_body(x_ref, gate_w_ref, sg_ref, su_ref, sd_ref,
                 sh_ref, logits_ref, ids_ref, pos0_ref, pos1_ref,
                 w0_ref, w1_ref, be_ref, par_ref):
    x = x_ref[...]
    logits = _dot_t(x, gate_w_ref[...])
    logits_ref[...] = logits
    m = jnp.max(logits, axis=1, keepdims=True)
    p = jnp.exp(logits - m)
    probs = p / jnp.sum(p, axis=1, keepdims=True)
    iota_e = lax.broadcasted_iota(jnp.int32, (T, E), 1)
    m1 = jnp.max(probs, axis=1, keepdims=True)
    a1 = jnp.min(jnp.where(probs == m1, iota_e, E), axis=1, keepdims=True)
    probs2 = jnp.where(iota_e == a1, -1.0, probs)
    m2 = jnp.max(probs2, axis=1, keepdims=True)
    a2 = jnp.min(jnp.where(probs2 == m2, iota_e, E), axis=1, keepdims=True)
    s = m1 + m2 + 1e-9
    ids_ref[...] = jnp.concatenate([a1, a2], axis=1)
    w0_ref[...] = jnp.broadcast_to(m1 / s, (T, 128))
    w1_ref[...] = jnp.broadcast_to(m2 / s, (T, 128))

    h1 = (iota_e == a1).astype(jnp.int32)
    h2 = (iota_e == a2).astype(jnp.int32)
    h = h1 + h2
    c = h
    sh = 1
    while sh < T:
        c = c + jnp.concatenate(
            [jnp.zeros((sh, E), jnp.int32), c[:T - sh]], axis=0)
        sh *= 2
    cexcl = c - h
    counts = c[T - 1:T, :]
    pc = ((counts + (BM - 1)) // BM) * BM
    cp = pc
    sh = 1
    while sh < E:
        cp = cp + jnp.concatenate(
            [jnp.zeros((1, sh), jnp.int32), cp[:, :E - sh]], axis=1)
        sh *= 2
    offs = cp - pc
    dest = offs + cexcl
    pos0_ref[...] = jnp.sum(jnp.where(iota_e == a1, dest, 0), axis=1,
                            keepdims=True)
    pos1_ref[...] = jnp.sum(jnp.where(iota_e == a2, dest, 0), axis=1,
                            keepdims=True)
    iota_b = lax.broadcasted_iota(jnp.int32, (NBLK, E), 0) * BM
    be = jnp.sum((iota_b >= jnp.broadcast_to(cp, (NBLK, E)))
                 .astype(jnp.int32), axis=1, keepdims=True)
    be_ref[...] = be
    trans = (be != jnp.concatenate([be[:1], be[:NBLK - 1]], axis=0)
             ).astype(jnp.int32)
    ct = trans
    sh2 = 1
    while sh2 < NBLK:
        ct = ct + jnp.concatenate(
            [jnp.zeros((sh2, 1), jnp.int32), ct[:NBLK - sh2]], axis=0)
        sh2 *= 2
    par_ref[...] = lax.rem(ct, 2)

    hg = _dot_t_bf16(x, sg_ref[...])
    hu = _dot_t_bf16(x, su_ref[...])
    sh_ref[...] = _dot_t_bf16(_silu(hg) * hu, sd_ref[...])


def _router(x, gate_w, sg, su, sd):
    return pl.pallas_call(
        _router_body,
        out_shape=[
            jax.ShapeDtypeStruct((T, D), jnp.float32),
            jax.ShapeDtypeStruct((T, E), jnp.float32),
            jax.ShapeDtypeStruct((T, K), jnp.int32),
            jax.ShapeDtypeStruct((T, 1), jnp.int32),
            jax.ShapeDtypeStruct((T, 1), jnp.int32),
            jax.ShapeDtypeStruct((T, 128), jnp.float32),
            jax.ShapeDtypeStruct((T, 128), jnp.float32),
            jax.ShapeDtypeStruct((NBLK, 1), jnp.int32),
            jax.ShapeDtypeStruct((NBLK, 1), jnp.int32),
        ],
    )(x, gate_w, sg, su, sd)


def _dispatch(x, pos0r, pos1r, w0m, w1m):
    mesh = plsc.VectorSubcoreMesh(core_axis_name="c", subcore_axis_name="s")

    @functools.partial(
        pl.kernel, mesh=mesh,
        out_type=[jax.ShapeDtypeStruct((S, D), jnp.float32),
                  jax.ShapeDtypeStruct((S, 128), jnp.float32)],
        scratch_types=[
            pltpu.VMEM((NCHUNK, CH), jnp.int32),
            pltpu.VMEM((NCHUNK, CH), jnp.int32),
            pltpu.VMEM((NCHUNK, CH, D), jnp.float32),
            pltpu.VMEM((NCHUNK, CH, 128), jnp.float32),
            pltpu.VMEM((NCHUNK, CH, 128), jnp.float32),
            pltpu.SemaphoreType.DMA,
        ],
    )
    def k(x_hbm, pos0_hbm, pos1_hbm, w0_hbm, w1_hbm, xs_hbm, ws_hbm,
          i0_v, i1_v, xr_v, w0_v, w1_v, sem):
        wid = lax.axis_index("s") * NC + lax.axis_index("c")
        handles = []
        for c in range(NCHUNK):
            row = wid * NCHUNK + c
            base = row * CH
            pltpu.sync_copy(pos0_hbm.at[row], i0_v.at[c])
            pltpu.sync_copy(pos1_hbm.at[row], i1_v.at[c])
            pltpu.sync_copy(x_hbm.at[pl.ds(base, CH)], xr_v.at[c])
            pltpu.sync_copy(w0_hbm.at[pl.ds(base, CH)], w0_v.at[c])
            pltpu.sync_copy(w1_hbm.at[pl.ds(base, CH)], w1_v.at[c])
            handles.append(
                pltpu.async_copy(xr_v.at[c], xs_hbm.at[i0_v.at[c]], sem))
            handles.append(
                pltpu.async_copy(xr_v.at[c], xs_hbm.at[i1_v.at[c]], sem))
            handles.append(
                pltpu.async_copy(w0_v.at[c], ws_hbm.at[i0_v.at[c]], sem))
            handles.append(
                pltpu.async_copy(w1_v.at[c], ws_hbm.at[i1_v.at[c]], sem))
        for h in handles:
            h.wait()

    return k(x, pos0r, pos1r, w0m, w1m)


def _gmlp_body(be_ref, par_ref, xs_ref, ws_ref, gp_hbm, up_hbm, dp_hbm,
               po_ref, gp_v, up_v, dp_v, sems):
    b = pl.program_id(0)
    e = be_ref[b]
    ec = jnp.minimum(e, E - 1)
    par = par_ref[b]

    def _start(exp, p):
        pltpu.make_async_copy(gp_hbm.at[exp], gp_v.at[p], sems.at[p]).start()
        pltpu.make_async_copy(up_hbm.at[exp], up_v.at[p], sems.at[p]).start()
        pltpu.make_async_copy(dp_hbm.at[exp], dp_v.at[p], sems.at[p]).start()

    def _wait(p):
        pltpu.make_async_copy(gp_hbm.at[0], gp_v.at[p], sems.at[p]).wait()
        pltpu.make_async_copy(up_hbm.at[0], up_v.at[p], sems.at[p]).wait()
        pltpu.make_async_copy(dp_hbm.at[0], dp_v.at[p], sems.at[p]).wait()

    @pl.when(b == 0)
    def _():
        _start(ec, par)

    first = jnp.logical_or(b == 0, e != be_ref[jnp.maximum(b - 1, 0)])

    @pl.when(jnp.logical_and(first, e < E))
    def _():
        _wait(par)

    nxt = be_ref[jnp.minimum(b + 1, NBLK - 1)]

    @pl.when(jnp.logical_and(b + 1 < NBLK,
                             jnp.logical_and(nxt != e, nxt < E)))
    def _():
        _start(nxt, 1 - par)

    @pl.when(e < E)
    def _():
        xb = xs_ref[...]
        hg = _dot_t_bf16(xb, gp_v.at[par][...])
        hu = _dot_t_bf16(xb, up_v.at[par][...])
        h = _silu(hg) * hu
        po_ref[...] = ws_ref[:, 0:1] * _dot_t_bf16(h, dp_v.at[par][...])


def _gmlp(blk_exp, blk_par, xs, ws, gp, up, dp):
    grid_spec = pltpu.PrefetchScalarGridSpec(
        num_scalar_prefetch=2,
        grid=(NBLK,),
        in_specs=[
            pl.BlockSpec((BM, D), lambda b, be, par: (b, 0)),
            pl.BlockSpec((BM, 128), lambda b, be, par: (b, 0)),
            pl.BlockSpec(memory_space=pl.ANY),
            pl.BlockSpec(memory_space=pl.ANY),
            pl.BlockSpec(memory_space=pl.ANY),
        ],
        out_specs=pl.BlockSpec((BM, D), lambda b, be, par: (b, 0)),
        scratch_shapes=[
            pltpu.VMEM((2, I, D), jnp.float32),
            pltpu.VMEM((2, I, D), jnp.float32),
            pltpu.VMEM((2, D, I), jnp.float32),
            pltpu.SemaphoreType.DMA((2,)),
        ],
    )
    return pl.pallas_call(
        _gmlp_body, grid_spec=grid_spec,
        out_shape=jax.ShapeDtypeStruct((S, D), jnp.float32),
    )(blk_exp, blk_par, xs, ws, gp, up, dp)


CCH = 16
CNCH = T // (NW * CCH)


def _combine(po, pos0r, pos1r, sh):
    mesh = plsc.VectorSubcoreMesh(core_axis_name="c", subcore_axis_name="s")

    @functools.partial(
        pl.kernel, mesh=mesh,
        out_type=jax.ShapeDtypeStruct((T, D), jnp.float32),
        scratch_types=[
            pltpu.VMEM((2, CCH), jnp.int32),
            pltpu.VMEM((2, CCH), jnp.int32),
            pltpu.VMEM((CCH, D), jnp.float32),
            pltpu.VMEM((2, CCH, D), jnp.float32),
            pltpu.VMEM((2, CCH, D), jnp.float32),
            pltpu.SemaphoreType.DMA,
            pltpu.SemaphoreType.DMA,
        ],
    )
    def k(po_hbm, pos0_hbm, pos1_hbm, sh_hbm, out_hbm,
          i0_v, i1_v, acc_v, g0_v, g1_v, sem_a, sem_b):
        wid = lax.axis_index("s") * NC + lax.axis_index("c")
        nv = D // 16
        sems = (sem_a, sem_b)

        def fire(c):
            pr = c % 2
            row = wid * CNCH + c
            pltpu.sync_copy(pos0_hbm.at[row], i0_v.at[pr])
            pltpu.sync_copy(pos1_hbm.at[row], i1_v.at[pr])
            return (pltpu.async_copy(po_hbm.at[i0_v.at[pr]], g0_v.at[pr],
                                     sems[pr]),
                    pltpu.async_copy(po_hbm.at[i1_v.at[pr]], g1_v.at[pr],
                                     sems[pr]))

        pending = fire(0)
        for c in range(CNCH):
            pr = c % 2
            base = (wid * CNCH + c) * CCH
            pltpu.sync_copy(sh_hbm.at[pl.ds(base, CCH)], acc_v)
            h0, h1 = pending
            h0.wait()
            h1.wait()
            if c + 1 < CNCH:
                pending = fire(c + 1)

            def addrow(r, carry):
                for v in range(nv):
                    sl = pl.ds(v * 16, 16)
                    plsc.addupdate(acc_v.at[r, sl],
                                   g0_v[pr, r, sl] + g1_v[pr, r, sl])
                return carry

            lax.fori_loop(0, CCH, addrow, 0)
            pltpu.sync_copy(acc_v, out_hbm.at[pl.ds(base, CCH)])

    return k(po, pos0r, pos1r, sh)


def kernel(hidden_state, gate_w, gate_proj, up_proj, down_proj, shared_gate,
           shared_up, shared_down):
    Bv, Nv, Dv = hidden_state.shape
    x = hidden_state.reshape(Bv * Nv, Dv)
    sh, logits, ids, pos0, pos1, w0m, w1m, be, bpar = _router(
        x, gate_w, shared_gate, shared_up, shared_down)
    xs, ws = _dispatch(x, pos0.reshape(T // CH, CH),
                       pos1.reshape(T // CH, CH), w0m, w1m)
    po = _gmlp(be.reshape(NBLK), bpar.reshape(NBLK), xs, ws,
               gate_proj, up_proj, down_proj)
    out = _combine(po, pos0.reshape(T // CCH, CCH),
                   pos1.reshape(T // CCH, CCH), sh)
    return out.reshape(Bv, Nv, Dv), logits, ids


# speedup vs baseline: 1.0768x; 1.0768x over previous
import functools

import jax
import jax.numpy as jnp
from jax import lax
from jax.experimental import pallas as pl
from jax.experimental.pallas import tpu as pltpu
from jax.experimental.pallas import tpu_sc as plsc

T, D, I, E, K = 2048, 1024, 512, 8, 2
BM = 256
S = 6144
NBLK = S // BM
NC, NS = 2, 16
NW = NC * NS
CH = 32
NCHUNK = T // (NW * CH)


def _dot_t(a, b):
    return lax.dot_general(a, b, (((1,), (1,)), ((), ())))


def _dot_t_bf16(a, b):
    return lax.dot_general(a.astype(jnp.bfloat16), b.astype(jnp.bfloat16),
                           (((1,), (1,)), ((), ())),
                           preferred_element_type=jnp.float32)


def _silu(x):
    return x / (1.0 + jnp.exp(-x))


def _router_body(x_ref, gate_w_ref, sg_ref, su_ref, sd_ref,
                 sh_ref, logits_ref, ids_ref, pos0_ref, pos1_ref,
                 w0_ref, w1_ref, be_ref, par_ref, bidx_ref, xbf_ref):
    x = x_ref[...]
    lo = x[:, :D // 2].astype(jnp.bfloat16).astype(jnp.float32)
    hi = x[:, D // 2:].astype(jnp.bfloat16).astype(jnp.float32)
    xbf_ref[...] = ((lax.bitcast_convert_type(lo, jnp.uint32) >> 16)
                    | (lax.bitcast_convert_type(hi, jnp.uint32)
                       & jnp.uint32(0xFFFF0000)))
    logits = _dot_t(x, gate_w_ref[...])
    logits_ref[...] = logits
    m = jnp.max(logits, axis=1, keepdims=True)
    p = jnp.exp(logits - m)
    probs = p / jnp.sum(p, axis=1, keepdims=True)
    iota_e = lax.broadcasted_iota(jnp.int32, (T, E), 1)
    m1 = jnp.max(probs, axis=1, keepdims=True)
    a1 = jnp.min(jnp.where(probs == m1, iota_e, E), axis=1, keepdims=True)
    probs2 = jnp.where(iota_e == a1, -1.0, probs)
    m2 = jnp.max(probs2, axis=1, keepdims=True)
    a2 = jnp.min(jnp.where(probs2 == m2, iota_e, E), axis=1, keepdims=True)
    s = m1 + m2 + 1e-9
    ids_ref[...] = jnp.concatenate([a1, a2], axis=1)
    w0_ref[...] = jnp.broadcast_to(m1 / s, (T, 128))
    w1_ref[...] = jnp.broadcast_to(m2 / s, (T, 128))

    h1 = (iota_e == a1).astype(jnp.int32)
    h2 = (iota_e == a2).astype(jnp.int32)
    h = h1 + h2
    c = h
    sh = 1
    while sh < T:
        c = c + jnp.concatenate(
            [jnp.zeros((sh, E), jnp.int32), c[:T - sh]], axis=0)
        sh *= 2
    cexcl = c - h
    counts = c[T - 1:T, :]
    pc = ((counts + (BM - 1)) // BM) * BM
    cp = pc
    sh = 1
    while sh < E:
        cp = cp + jnp.concatenate(
            [jnp.zeros((1, sh), jnp.int32), cp[:, :E - sh]], axis=1)
        sh *= 2
    offs = cp - pc
    dest = offs + cexcl
    pos0_ref[...] = jnp.sum(jnp.where(iota_e == a1, dest, 0), axis=1,
                            keepdims=True)
    pos1_ref[...] = jnp.sum(jnp.where(iota_e == a2, dest, 0), axis=1,
                            keepdims=True)
    iota_b = lax.broadcasted_iota(jnp.int32, (NBLK, E), 0) * BM
    be = jnp.sum((iota_b >= jnp.broadcast_to(cp, (NBLK, E)))
                 .astype(jnp.int32), axis=1, keepdims=True)
    be_ref[...] = be
    trans = (be != jnp.concatenate([be[:1], be[:NBLK - 1]], axis=0)
             ).astype(jnp.int32)
    ct = trans
    sh2 = 1
    while sh2 < NBLK:
        ct = ct + jnp.concatenate(
            [jnp.zeros((sh2, 1), jnp.int32), ct[:NBLK - sh2]], axis=0)
        sh2 *= 2
    par_ref[...] = lax.rem(ct, 2)
    lastr = cp[:, E - 1:E] // BM - 1
    iota_r = lax.broadcasted_iota(jnp.int32, (NBLK, 1), 0)
    bidx_ref[...] = jnp.minimum(iota_r, jnp.broadcast_to(lastr, (NBLK, 1)))

    hg = _dot_t_bf16(x, sg_ref[...])
    hu = _dot_t_bf16(x, su_ref[...])
    sh_ref[...] = _dot_t_bf16(_silu(hg) * hu, sd_ref[...])


def _router(x, gate_w, sg, su, sd):
    return pl.pallas_call(
        _router_body,
        out_shape=[
            jax.ShapeDtypeStruct((T, D), jnp.float32),
            jax.ShapeDtypeStruct((T, E), jnp.float32),
            jax.ShapeDtypeStruct((T, K), jnp.int32),
            jax.ShapeDtypeStruct((T, 1), jnp.int32),
            jax.ShapeDtypeStruct((T, 1), jnp.int32),
            jax.ShapeDtypeStruct((T, 128), jnp.float32),
            jax.ShapeDtypeStruct((T, 128), jnp.float32),
            jax.ShapeDtypeStruct((NBLK, 1), jnp.int32),
            jax.ShapeDtypeStruct((NBLK, 1), jnp.int32),
            jax.ShapeDtypeStruct((NBLK, 1), jnp.int32),
            jax.ShapeDtypeStruct((T, D // 2), jnp.uint32),
        ],
    )(x, gate_w, sg, su, sd)


def _dispatch(x, pos0r, pos1r, w0m, w1m):
    mesh = plsc.VectorSubcoreMesh(core_axis_name="c", subcore_axis_name="s")

    @functools.partial(
        pl.kernel, mesh=mesh,
        out_type=[jax.ShapeDtypeStruct((S, D // 2), jnp.uint32),
                  jax.ShapeDtypeStruct((S, 128), jnp.float32)],
        scratch_types=[
            pltpu.VMEM((NCHUNK, CH), jnp.int32),
            pltpu.VMEM((NCHUNK, CH), jnp.int32),
            pltpu.VMEM((NCHUNK, CH, D // 2), jnp.uint32),
            pltpu.VMEM((NCHUNK, CH, 128), jnp.float32),
            pltpu.VMEM((NCHUNK, CH, 128), jnp.float32),
            pltpu.SemaphoreType.DMA,
        ],
    )
    def k(x_hbm, pos0_hbm, pos1_hbm, w0_hbm, w1_hbm, xs_hbm, ws_hbm,
          i0_v, i1_v, xr_v, w0_v, w1_v, sem):
        wid = lax.axis_index("s") * NC + lax.axis_index("c")
        handles = []
        for c in range(NCHUNK):
            row = wid * NCHUNK + c
            base = row * CH
            pltpu.sync_copy(pos0_hbm.at[row], i0_v.at[c])
            pltpu.sync_copy(pos1_hbm.at[row], i1_v.at[c])
            pltpu.sync_copy(x_hbm.at[pl.ds(base, CH)], xr_v.at[c])
            pltpu.sync_copy(w0_hbm.at[pl.ds(base, CH)], w0_v.at[c])
            pltpu.sync_copy(w1_hbm.at[pl.ds(base, CH)], w1_v.at[c])
            handles.append(
                pltpu.async_copy(xr_v.at[c], xs_hbm.at[i0_v.at[c]], sem))
            handles.append(
                pltpu.async_copy(xr_v.at[c], xs_hbm.at[i1_v.at[c]], sem))
            handles.append(
                pltpu.async_copy(w0_v.at[c], ws_hbm.at[i0_v.at[c]], sem))
            handles.append(
                pltpu.async_copy(w1_v.at[c], ws_hbm.at[i1_v.at[c]], sem))
        for h in handles:
            h.wait()

    return k(x, pos0r, pos1r, w0m, w1m)


def _gmlp_body(be_ref, par_ref, bidx_ref, xs_ref, ws_ref, gp_hbm, up_hbm,
               dp_hbm, po_ref, gp_v, up_v, dp_v, sems):
    b = pl.program_id(0)
    e = be_ref[b]
    ec = jnp.minimum(e, E - 1)
    par = par_ref[b]

    def _start(exp, p):
        pltpu.make_async_copy(gp_hbm.at[exp], gp_v.at[p], sems.at[p]).start()
        pltpu.make_async_copy(up_hbm.at[exp], up_v.at[p], sems.at[p]).start()
        pltpu.make_async_copy(dp_hbm.at[exp], dp_v.at[p], sems.at[p]).start()

    def _wait(p):
        pltpu.make_async_copy(gp_hbm.at[0], gp_v.at[p], sems.at[p]).wait()
        pltpu.make_async_copy(up_hbm.at[0], up_v.at[p], sems.at[p]).wait()
        pltpu.make_async_copy(dp_hbm.at[0], dp_v.at[p], sems.at[p]).wait()

    @pl.when(b == 0)
    def _():
        _start(ec, par)

    first = jnp.logical_or(b == 0, e != be_ref[jnp.maximum(b - 1, 0)])

    @pl.when(jnp.logical_and(first, e < E))
    def _():
        _wait(par)

    nxt = be_ref[jnp.minimum(b + 1, NBLK - 1)]

    @pl.when(jnp.logical_and(b + 1 < NBLK,
                             jnp.logical_and(nxt != e, nxt < E)))
    def _():
        _start(nxt, 1 - par)

    @pl.when(e < E)
    def _():
        w = xs_ref[...]
        xlo = lax.bitcast_convert_type(w << 16, jnp.float32)
        xhi = lax.bitcast_convert_type(w & jnp.uint32(0xFFFF0000),
                                       jnp.float32)
        xb = jnp.concatenate([xlo, xhi], axis=1)
        hg = _dot_t_bf16(xb, gp_v.at[par][...])
        hu = _dot_t_bf16(xb, up_v.at[par][...])
        h = _silu(hg) * hu
        po_ref[...] = ws_ref[:, 0:1] * _dot_t_bf16(h, dp_v.at[par][...])


def _gmlp(blk_exp, blk_par, blk_idx, xs, ws, gp, up, dp):
    def _bmap(b, be, par, bidx):
        return (bidx[b], 0)

    grid_spec = pltpu.PrefetchScalarGridSpec(
        num_scalar_prefetch=3,
        grid=(NBLK,),
        in_specs=[
            pl.BlockSpec((BM, D // 2), _bmap),
            pl.BlockSpec((BM, 128), _bmap),
            pl.BlockSpec(memory_space=pl.ANY),
            pl.BlockSpec(memory_space=pl.ANY),
            pl.BlockSpec(memory_space=pl.ANY),
        ],
        out_specs=pl.BlockSpec((BM, D), _bmap),
        scratch_shapes=[
            pltpu.VMEM((2, I, D), jnp.float32),
            pltpu.VMEM((2, I, D), jnp.float32),
            pltpu.VMEM((2, D, I), jnp.float32),
            pltpu.SemaphoreType.DMA((2,)),
        ],
    )
    return pl.pallas_call(
        _gmlp_body, grid_spec=grid_spec,
        out_shape=jax.ShapeDtypeStruct((S, D), jnp.float32),
    )(blk_exp, blk_par, blk_idx, xs, ws, gp, up, dp)


CCH = 16
CNCH = T // (NW * CCH)


def _combine(po, pos0r, pos1r, sh):
    mesh = plsc.VectorSubcoreMesh(core_axis_name="c", subcore_axis_name="s")

    @functools.partial(
        pl.kernel, mesh=mesh,
        out_type=jax.ShapeDtypeStruct((T, D), jnp.float32),
        scratch_types=[
            pltpu.VMEM((2, CCH), jnp.int32),
            pltpu.VMEM((2, CCH), jnp.int32),
            pltpu.VMEM((CCH, D), jnp.float32),
            pltpu.VMEM((2, CCH, D), jnp.float32),
            pltpu.VMEM((2, CCH, D), jnp.float32),
            pltpu.SemaphoreType.DMA,
            pltpu.SemaphoreType.DMA,
        ],
    )
    def k(po_hbm, pos0_hbm, pos1_hbm, sh_hbm, out_hbm,
          i0_v, i1_v, acc_v, g0_v, g1_v, sem_a, sem_b):
        wid = lax.axis_index("s") * NC + lax.axis_index("c")
        nv = D // 16
        sems = (sem_a, sem_b)

        def fire(c):
            pr = c % 2
            row = wid * CNCH + c
            pltpu.sync_copy(pos0_hbm.at[row], i0_v.at[pr])
            pltpu.sync_copy(pos1_hbm.at[row], i1_v.at[pr])
            return (pltpu.async_copy(po_hbm.at[i0_v.at[pr]], g0_v.at[pr],
                                     sems[pr]),
                    pltpu.async_copy(po_hbm.at[i1_v.at[pr]], g1_v.at[pr],
                                     sems[pr]))

        pending = fire(0)
        for c in range(CNCH):
            pr = c % 2
            base = (wid * CNCH + c) * CCH
            pltpu.sync_copy(sh_hbm.at[pl.ds(base, CCH)], acc_v)
            h0, h1 = pending
            h0.wait()
            h1.wait()
            if c + 1 < CNCH:
                pending = fire(c + 1)

            def addrow(r, carry):
                for v in range(nv):
                    sl = pl.ds(v * 16, 16)
                    plsc.addupdate(acc_v.at[r, sl],
                                   g0_v[pr, r, sl] + g1_v[pr, r, sl])
                return carry

            lax.fori_loop(0, CCH, addrow, 0)
            pltpu.sync_copy(acc_v, out_hbm.at[pl.ds(base, CCH)])

    return k(po, pos0r, pos1r, sh)


def kernel(hidden_state, gate_w, gate_proj, up_proj, down_proj, shared_gate,
           shared_up, shared_down):
    Bv, Nv, Dv = hidden_state.shape
    x = hidden_state.reshape(Bv * Nv, Dv)
    sh, logits, ids, pos0, pos1, w0m, w1m, be, bpar, bidx, xbf = _router(
        x, gate_w, shared_gate, shared_up, shared_down)
    xs, ws = _dispatch(xbf, pos0.reshape(T // CH, CH),
                       pos1.reshape(T // CH, CH), w0m, w1m)
    po = _gmlp(be.reshape(NBLK), bpar.reshape(NBLK), bidx.reshape(NBLK),
               xs, ws, gate_proj, up_proj, down_proj)
    out = _combine(po, pos0.reshape(T // CCH, CCH),
                   pos1.reshape(T // CCH, CCH), sh)
    return out.reshape(Bv, Nv, Dv), logits, ids


# speedup vs baseline: 1.0803x; 1.0033x over previous
import functools

import jax
import jax.numpy as jnp
from jax import lax
from jax.experimental import pallas as pl
from jax.experimental.pallas import tpu as pltpu
from jax.experimental.pallas import tpu_sc as plsc

T, D, I, E, K = 2048, 1024, 512, 8, 2
BM = 256
S = 6144
NBLK = S // BM
NC, NS = 2, 16
NW = NC * NS
CH = 32
NCHUNK = T // (NW * CH)


def _dot_t(a, b):
    return lax.dot_general(a, b, (((1,), (1,)), ((), ())))


def _dot_t_bf16(a, b):
    return lax.dot_general(a.astype(jnp.bfloat16), b.astype(jnp.bfloat16),
                           (((1,), (1,)), ((), ())),
                           preferred_element_type=jnp.float32)


def _silu(x):
    return x / (1.0 + jnp.exp(-x))


def _router_body(x_ref, gate_w_ref, sg_ref, su_ref, sd_ref,
                 sh_ref, logits_ref, ids_ref, pos0_ref, pos1_ref,
                 w0_ref, w1_ref, be_ref, par_ref, bidx_ref, xbf_ref):
    x = x_ref[...]
    lo = x[:, :D // 2].astype(jnp.bfloat16).astype(jnp.float32)
    hi = x[:, D // 2:].astype(jnp.bfloat16).astype(jnp.float32)
    xbf_ref[...] = (((lax.bitcast_convert_type(lo, jnp.int32) >> 16)
                     & jnp.int32(0xFFFF))
                    | (lax.bitcast_convert_type(hi, jnp.int32)
                       & jnp.int32(-65536)))
    logits = _dot_t(x, gate_w_ref[...])
    logits_ref[...] = logits
    m = jnp.max(logits, axis=1, keepdims=True)
    p = jnp.exp(logits - m)
    probs = p / jnp.sum(p, axis=1, keepdims=True)
    iota_e = lax.broadcasted_iota(jnp.int32, (T, E), 1)
    m1 = jnp.max(probs, axis=1, keepdims=True)
    a1 = jnp.min(jnp.where(probs == m1, iota_e, E), axis=1, keepdims=True)
    probs2 = jnp.where(iota_e == a1, -1.0, probs)
    m2 = jnp.max(probs2, axis=1, keepdims=True)
    a2 = jnp.min(jnp.where(probs2 == m2, iota_e, E), axis=1, keepdims=True)
    s = m1 + m2 + 1e-9
    ids_ref[...] = jnp.concatenate([a1, a2], axis=1)
    w0_ref[...] = jnp.broadcast_to(m1 / s, (T, 128))
    w1_ref[...] = jnp.broadcast_to(m2 / s, (T, 128))

    h1 = (iota_e == a1).astype(jnp.int32)
    h2 = (iota_e == a2).astype(jnp.int32)
    h = h1 + h2
    c = h
    sh = 1
    while sh < T:
        c = c + jnp.concatenate(
            [jnp.zeros((sh, E), jnp.int32), c[:T - sh]], axis=0)
        sh *= 2
    cexcl = c - h
    counts = c[T - 1:T, :]
    pc = ((counts + (BM - 1)) // BM) * BM
    cp = pc
    sh = 1
    while sh < E:
        cp = cp + jnp.concatenate(
            [jnp.zeros((1, sh), jnp.int32), cp[:, :E - sh]], axis=1)
        sh *= 2
    offs = cp - pc
    dest = offs + cexcl
    pos0_ref[...] = jnp.sum(jnp.where(iota_e == a1, dest, 0), axis=1,
                            keepdims=True)
    pos1_ref[...] = jnp.sum(jnp.where(iota_e == a2, dest, 0), axis=1,
                            keepdims=True)
    iota_b = lax.broadcasted_iota(jnp.int32, (NBLK, E), 0) * BM
    be = jnp.sum((iota_b >= jnp.broadcast_to(cp, (NBLK, E)))
                 .astype(jnp.int32), axis=1, keepdims=True)
    be_ref[...] = be
    trans = (be != jnp.concatenate([be[:1], be[:NBLK - 1]], axis=0)
             ).astype(jnp.int32)
    ct = trans
    sh2 = 1
    while sh2 < NBLK:
        ct = ct + jnp.concatenate(
            [jnp.zeros((sh2, 1), jnp.int32), ct[:NBLK - sh2]], axis=0)
        sh2 *= 2
    par_ref[...] = lax.rem(ct, 2)
    lastr = cp[:, E - 1:E] // BM - 1
    iota_r = lax.broadcasted_iota(jnp.int32, (NBLK, 1), 0)
    bidx_ref[...] = jnp.minimum(iota_r, jnp.broadcast_to(lastr, (NBLK, 1)))

    hg = _dot_t_bf16(x, sg_ref[...])
    hu = _dot_t_bf16(x, su_ref[...])
    sh_ref[...] = _dot_t_bf16(_silu(hg) * hu, sd_ref[...])


def _router(x, gate_w, sg, su, sd):
    return pl.pallas_call(
        _router_body,
        out_shape=[
            jax.ShapeDtypeStruct((T, D), jnp.float32),
            jax.ShapeDtypeStruct((T, E), jnp.float32),
            jax.ShapeDtypeStruct((T, K), jnp.int32),
            jax.ShapeDtypeStruct((T, 1), jnp.int32),
            jax.ShapeDtypeStruct((T, 1), jnp.int32),
            jax.ShapeDtypeStruct((T, 128), jnp.float32),
            jax.ShapeDtypeStruct((T, 128), jnp.float32),
            jax.ShapeDtypeStruct((NBLK, 1), jnp.int32),
            jax.ShapeDtypeStruct((NBLK, 1), jnp.int32),
            jax.ShapeDtypeStruct((NBLK, 1), jnp.int32),
            jax.ShapeDtypeStruct((T, D // 2), jnp.int32),
        ],
    )(x, gate_w, sg, su, sd)


def _dispatch(x, pos0r, pos1r, w0m, w1m):
    mesh = plsc.VectorSubcoreMesh(core_axis_name="c", subcore_axis_name="s")

    @functools.partial(
        pl.kernel, mesh=mesh,
        out_type=[jax.ShapeDtypeStruct((S, D // 2), jnp.int32),
                  jax.ShapeDtypeStruct((S, 128), jnp.float32)],
        scratch_types=[
            pltpu.VMEM((NCHUNK, CH), jnp.int32),
            pltpu.VMEM((NCHUNK, CH), jnp.int32),
            pltpu.VMEM((NCHUNK, CH, D // 2), jnp.int32),
            pltpu.VMEM((NCHUNK, CH, 128), jnp.float32),
            pltpu.VMEM((NCHUNK, CH, 128), jnp.float32),
            pltpu.SemaphoreType.DMA,
        ],
    )
    def k(x_hbm, pos0_hbm, pos1_hbm, w0_hbm, w1_hbm, xs_hbm, ws_hbm,
          i0_v, i1_v, xr_v, w0_v, w1_v, sem):
        wid = lax.axis_index("s") * NC + lax.axis_index("c")
        handles = []
        for c in range(NCHUNK):
            row = wid * NCHUNK + c
            base = row * CH
            pltpu.sync_copy(pos0_hbm.at[row], i0_v.at[c])
            pltpu.sync_copy(pos1_hbm.at[row], i1_v.at[c])
            pltpu.sync_copy(x_hbm.at[pl.ds(base, CH)], xr_v.at[c])
            pltpu.sync_copy(w0_hbm.at[pl.ds(base, CH)], w0_v.at[c])
            pltpu.sync_copy(w1_hbm.at[pl.ds(base, CH)], w1_v.at[c])
            handles.append(
                pltpu.async_copy(xr_v.at[c], xs_hbm.at[i0_v.at[c]], sem))
            handles.append(
                pltpu.async_copy(xr_v.at[c], xs_hbm.at[i1_v.at[c]], sem))
            handles.append(
                pltpu.async_copy(w0_v.at[c], ws_hbm.at[i0_v.at[c]], sem))
            handles.append(
                pltpu.async_copy(w1_v.at[c], ws_hbm.at[i1_v.at[c]], sem))
        for h in handles:
            h.wait()

    return k(x, pos0r, pos1r, w0m, w1m)


def _gmlp_body(be_ref, par_ref, bidx_ref, xs_ref, ws_ref, gp_hbm, up_hbm,
               dp_hbm, po_ref, gp_v, up_v, dp_v, sems):
    b = pl.program_id(0)
    e = be_ref[b]
    ec = jnp.minimum(e, E - 1)
    par = par_ref[b]

    def _start(exp, p):
        pltpu.make_async_copy(gp_hbm.at[exp], gp_v.at[p], sems.at[p]).start()
        pltpu.make_async_copy(up_hbm.at[exp], up_v.at[p], sems.at[p]).start()
        pltpu.make_async_copy(dp_hbm.at[exp], dp_v.at[p], sems.at[p]).start()

    def _wait(p):
        pltpu.make_async_copy(gp_hbm.at[0], gp_v.at[p], sems.at[p]).wait()
        pltpu.make_async_copy(up_hbm.at[0], up_v.at[p], sems.at[p]).wait()
        pltpu.make_async_copy(dp_hbm.at[0], dp_v.at[p], sems.at[p]).wait()

    @pl.when(b == 0)
    def _():
        _start(ec, par)

    first = jnp.logical_or(b == 0, e != be_ref[jnp.maximum(b - 1, 0)])

    @pl.when(jnp.logical_and(first, e < E))
    def _():
        _wait(par)

    nxt = be_ref[jnp.minimum(b + 1, NBLK - 1)]

    @pl.when(jnp.logical_and(b + 1 < NBLK,
                             jnp.logical_and(nxt != e, nxt < E)))
    def _():
        _start(nxt, 1 - par)

    @pl.when(e < E)
    def _():
        w = xs_ref[...]
        xlo = lax.bitcast_convert_type(w << 16, jnp.float32)
        xhi = lax.bitcast_convert_type(w & jnp.int32(-65536), jnp.float32)
        xb = jnp.concatenate([xlo, xhi], axis=1)
        hg = _dot_t_bf16(xb, gp_v.at[par][...])
        hu = _dot_t_bf16(xb, up_v.at[par][...])
        h = _silu(hg) * hu
        eo = ws_ref[:, 0:1] * _dot_t_bf16(h, dp_v.at[par][...])
        elo = eo[:, :D // 2].astype(jnp.bfloat16).astype(jnp.float32)
        ehi = eo[:, D // 2:].astype(jnp.bfloat16).astype(jnp.float32)
        po_ref[...] = (((lax.bitcast_convert_type(elo, jnp.int32) >> 16)
                        & jnp.int32(0xFFFF))
                       | (lax.bitcast_convert_type(ehi, jnp.int32)
                          & jnp.int32(-65536)))


def _gmlp(blk_exp, blk_par, blk_idx, xs, ws, gp, up, dp):
    def _bmap(b, be, par, bidx):
        return (bidx[b], 0)

    grid_spec = pltpu.PrefetchScalarGridSpec(
        num_scalar_prefetch=3,
        grid=(NBLK,),
        in_specs=[
            pl.BlockSpec((BM, D // 2), _bmap),
            pl.BlockSpec((BM, 128), _bmap),
            pl.BlockSpec(memory_space=pl.ANY),
            pl.BlockSpec(memory_space=pl.ANY),
            pl.BlockSpec(memory_space=pl.ANY),
        ],
        out_specs=pl.BlockSpec((BM, D // 2), _bmap),
        scratch_shapes=[
            pltpu.VMEM((2, I, D), jnp.float32),
            pltpu.VMEM((2, I, D), jnp.float32),
            pltpu.VMEM((2, D, I), jnp.float32),
            pltpu.SemaphoreType.DMA((2,)),
        ],
    )
    return pl.pallas_call(
        _gmlp_body, grid_spec=grid_spec,
        out_shape=jax.ShapeDtypeStruct((S, D // 2), jnp.int32),
    )(blk_exp, blk_par, blk_idx, xs, ws, gp, up, dp)


CCH = 16
CNCH = T // (NW * CCH)


def _combine(po, pos0r, pos1r, sh):
    mesh = plsc.VectorSubcoreMesh(core_axis_name="c", subcore_axis_name="s")

    @functools.partial(
        pl.kernel, mesh=mesh,
        out_type=jax.ShapeDtypeStruct((T, D), jnp.float32),
        scratch_types=[
            pltpu.VMEM((2, CCH), jnp.int32),
            pltpu.VMEM((2, CCH), jnp.int32),
            pltpu.VMEM((CCH, D), jnp.float32),
            pltpu.VMEM((2, CCH, D // 2), jnp.int32),
            pltpu.VMEM((2, CCH, D // 2), jnp.int32),
            pltpu.SemaphoreType.DMA,
            pltpu.SemaphoreType.DMA,
        ],
    )
    def k(po_hbm, pos0_hbm, pos1_hbm, sh_hbm, out_hbm,
          i0_v, i1_v, acc_v, g0_v, g1_v, sem_a, sem_b):
        wid = lax.axis_index("s") * NC + lax.axis_index("c")
        nv = D // 16
        sems = (sem_a, sem_b)

        def fire(c):
            pr = c % 2
            row = wid * CNCH + c
            pltpu.sync_copy(pos0_hbm.at[row], i0_v.at[pr])
            pltpu.sync_copy(pos1_hbm.at[row], i1_v.at[pr])
            return (pltpu.async_copy(po_hbm.at[i0_v.at[pr]], g0_v.at[pr],
                                     sems[pr]),
                    pltpu.async_copy(po_hbm.at[i1_v.at[pr]], g1_v.at[pr],
                                     sems[pr]))

        pending = fire(0)
        for c in range(CNCH):
            pr = c % 2
            base = (wid * CNCH + c) * CCH
            pltpu.sync_copy(sh_hbm.at[pl.ds(base, CCH)], acc_v)
            h0, h1 = pending
            h0.wait()
            h1.wait()
            if c + 1 < CNCH:
                pending = fire(c + 1)

            def addrow(r, carry):
                for v in range(D // 32):
                    sl = pl.ds(v * 16, 16)
                    w0 = g0_v[pr, r, sl]
                    w1 = g1_v[pr, r, sl]
                    lo = (lax.bitcast_convert_type(w0 << 16, jnp.float32)
                          + lax.bitcast_convert_type(w1 << 16, jnp.float32))
                    hi = (lax.bitcast_convert_type(w0 & jnp.int32(-65536),
                                                   jnp.float32)
                          + lax.bitcast_convert_type(w1 & jnp.int32(-65536),
                                                     jnp.float32))
                    plsc.addupdate(acc_v.at[r, sl], lo)
                    plsc.addupdate(acc_v.at[r, pl.ds(D // 2 + v * 16, 16)],
                                   hi)
                return carry

            lax.fori_loop(0, CCH, addrow, 0)
            pltpu.sync_copy(acc_v, out_hbm.at[pl.ds(base, CCH)])

    return k(po, pos0r, pos1r, sh)


def kernel(hidden_state, gate_w, gate_proj, up_proj, down_proj, shared_gate,
           shared_up, shared_down):
    Bv, Nv, Dv = hidden_state.shape
    x = hidden_state.reshape(Bv * Nv, Dv)
    sh, logits, ids, pos0, pos1, w0m, w1m, be, bpar, bidx, xbf = _router(
        x, gate_w, shared_gate, shared_up, shared_down)
    xs, ws = _dispatch(xbf, pos0.reshape(T // CH, CH),
                       pos1.reshape(T // CH, CH), w0m, w1m)
    po = _gmlp(be.reshape(NBLK), bpar.reshape(NBLK), bidx.reshape(NBLK),
               xs, ws, gate_proj, up_proj, down_proj)
    out = _combine(po, pos0.reshape(T // CCH, CCH),
                   pos1.reshape(T // CCH, CCH), sh)
    return out.reshape(Bv, Nv, Dv), logits, ids


# speedup vs baseline: 1.0854x; 1.0047x over previous
import functools

import jax
import jax.numpy as jnp
from jax import lax
from jax.experimental import pallas as pl
from jax.experimental.pallas import tpu as pltpu
from jax.experimental.pallas import tpu_sc as plsc

T, D, I, E, K = 2048, 1024, 512, 8, 2
BM = 256
S = 6144
NBLK = S // BM
NC, NS = 2, 16
NW = NC * NS
CH = 32
NCHUNK = T // (NW * CH)


def _dot_t(a, b):
    return lax.dot_general(a, b, (((1,), (1,)), ((), ())))


def _dot_t_bf16(a, b):
    return lax.dot_general(a.astype(jnp.bfloat16), b.astype(jnp.bfloat16),
                           (((1,), (1,)), ((), ())),
                           preferred_element_type=jnp.float32)


def _silu(x):
    return x / (1.0 + jnp.exp(-x))


def _router_body(x_ref, gate_w_ref,
                 logits_ref, ids_ref, pos0_ref, pos1_ref,
                 w0_ref, w1_ref, be_ref, par_ref, bidx_ref, xbf_ref):
    x = x_ref[...]
    lo = x[:, :D // 2].astype(jnp.bfloat16).astype(jnp.float32)
    hi = x[:, D // 2:].astype(jnp.bfloat16).astype(jnp.float32)
    xbf_ref[...] = (((lax.bitcast_convert_type(lo, jnp.int32) >> 16)
                     & jnp.int32(0xFFFF))
                    | (lax.bitcast_convert_type(hi, jnp.int32)
                       & jnp.int32(-65536)))
    logits = _dot_t(x, gate_w_ref[...])
    logits_ref[...] = logits
    m = jnp.max(logits, axis=1, keepdims=True)
    p = jnp.exp(logits - m)
    probs = p / jnp.sum(p, axis=1, keepdims=True)
    iota_e = lax.broadcasted_iota(jnp.int32, (T, E), 1)
    m1 = jnp.max(probs, axis=1, keepdims=True)
    a1 = jnp.min(jnp.where(probs == m1, iota_e, E), axis=1, keepdims=True)
    probs2 = jnp.where(iota_e == a1, -1.0, probs)
    m2 = jnp.max(probs2, axis=1, keepdims=True)
    a2 = jnp.min(jnp.where(probs2 == m2, iota_e, E), axis=1, keepdims=True)
    s = m1 + m2 + 1e-9
    ids_ref[...] = jnp.concatenate([a1, a2], axis=1)
    w0_ref[...] = jnp.broadcast_to(m1 / s, (T, 128))
    w1_ref[...] = jnp.broadcast_to(m2 / s, (T, 128))

    h1 = (iota_e == a1).astype(jnp.int32)
    h2 = (iota_e == a2).astype(jnp.int32)
    h = h1 + h2
    c = h
    sh = 1
    while sh < T:
        c = c + jnp.concatenate(
            [jnp.zeros((sh, E), jnp.int32), c[:T - sh]], axis=0)
        sh *= 2
    cexcl = c - h
    counts = c[T - 1:T, :]
    pc = ((counts + (BM - 1)) // BM) * BM
    cp = pc
    sh = 1
    while sh < E:
        cp = cp + jnp.concatenate(
            [jnp.zeros((1, sh), jnp.int32), cp[:, :E - sh]], axis=1)
        sh *= 2
    offs = cp - pc
    dest = offs + cexcl
    pos0_ref[...] = jnp.sum(jnp.where(iota_e == a1, dest, 0), axis=1,
                            keepdims=True)
    pos1_ref[...] = jnp.sum(jnp.where(iota_e == a2, dest, 0), axis=1,
                            keepdims=True)
    iota_b = lax.broadcasted_iota(jnp.int32, (NBLK, E), 0) * BM
    be = jnp.sum((iota_b >= jnp.broadcast_to(cp, (NBLK, E)))
                 .astype(jnp.int32), axis=1, keepdims=True)
    be_ref[...] = be
    trans = (be != jnp.concatenate([be[:1], be[:NBLK - 1]], axis=0)
             ).astype(jnp.int32)
    ct = trans
    sh2 = 1
    while sh2 < NBLK:
        ct = ct + jnp.concatenate(
            [jnp.zeros((sh2, 1), jnp.int32), ct[:NBLK - sh2]], axis=0)
        sh2 *= 2
    par_ref[...] = lax.rem(ct, 2)
    lastr = cp[:, E - 1:E] // BM - 1
    iota_r = lax.broadcasted_iota(jnp.int32, (NBLK, 1), 0)
    bidx_ref[...] = jnp.minimum(iota_r, jnp.broadcast_to(lastr, (NBLK, 1)))


def _shared_body(x_ref, sg_ref, su_ref, sd_ref, sh_ref):
    x = x_ref[...]
    hg = _dot_t_bf16(x, sg_ref[...])
    hu = _dot_t_bf16(x, su_ref[...])
    sh_ref[...] = _dot_t_bf16(_silu(hg) * hu, sd_ref[...])


def _shared(x, sg, su, sd):
    return pl.pallas_call(
        _shared_body,
        grid=(T // 256,),
        in_specs=[
            pl.BlockSpec((256, D), lambda i: (i, 0)),
            pl.BlockSpec((I, D), lambda i: (0, 0)),
            pl.BlockSpec((I, D), lambda i: (0, 0)),
            pl.BlockSpec((D, I), lambda i: (0, 0)),
        ],
        out_specs=pl.BlockSpec((256, D), lambda i: (i, 0)),
        out_shape=jax.ShapeDtypeStruct((T, D), jnp.float32),
    )(x, sg, su, sd)


def _router(x, gate_w):
    return pl.pallas_call(
        _router_body,
        out_shape=[
            jax.ShapeDtypeStruct((T, E), jnp.float32),
            jax.ShapeDtypeStruct((T, K), jnp.int32),
            jax.ShapeDtypeStruct((T, 1), jnp.int32),
            jax.ShapeDtypeStruct((T, 1), jnp.int32),
            jax.ShapeDtypeStruct((T, 128), jnp.float32),
            jax.ShapeDtypeStruct((T, 128), jnp.float32),
            jax.ShapeDtypeStruct((NBLK, 1), jnp.int32),
            jax.ShapeDtypeStruct((NBLK, 1), jnp.int32),
            jax.ShapeDtypeStruct((NBLK, 1), jnp.int32),
            jax.ShapeDtypeStruct((T, D // 2), jnp.int32),
        ],
    )(x, gate_w)


def _dispatch(x, pos0r, pos1r, w0m, w1m):
    mesh = plsc.VectorSubcoreMesh(core_axis_name="c", subcore_axis_name="s")

    @functools.partial(
        pl.kernel, mesh=mesh,
        out_type=[jax.ShapeDtypeStruct((S, D // 2), jnp.int32),
                  jax.ShapeDtypeStruct((S, 128), jnp.float32)],
        scratch_types=[
            pltpu.VMEM((NCHUNK, CH), jnp.int32),
            pltpu.VMEM((NCHUNK, CH), jnp.int32),
            pltpu.VMEM((NCHUNK, CH, D // 2), jnp.int32),
            pltpu.VMEM((NCHUNK, CH, 128), jnp.float32),
            pltpu.VMEM((NCHUNK, CH, 128), jnp.float32),
            pltpu.SemaphoreType.DMA,
        ],
    )
    def k(x_hbm, pos0_hbm, pos1_hbm, w0_hbm, w1_hbm, xs_hbm, ws_hbm,
          i0_v, i1_v, xr_v, w0_v, w1_v, sem):
        wid = lax.axis_index("s") * NC + lax.axis_index("c")
        handles = []
        for c in range(NCHUNK):
            row = wid * NCHUNK + c
            base = row * CH
            pltpu.sync_copy(pos0_hbm.at[row], i0_v.at[c])
            pltpu.sync_copy(pos1_hbm.at[row], i1_v.at[c])
            pltpu.sync_copy(x_hbm.at[pl.ds(base, CH)], xr_v.at[c])
            pltpu.sync_copy(w0_hbm.at[pl.ds(base, CH)], w0_v.at[c])
            pltpu.sync_copy(w1_hbm.at[pl.ds(base, CH)], w1_v.at[c])
            handles.append(
                pltpu.async_copy(xr_v.at[c], xs_hbm.at[i0_v.at[c]], sem))
            handles.append(
                pltpu.async_copy(xr_v.at[c], xs_hbm.at[i1_v.at[c]], sem))
            handles.append(
                pltpu.async_copy(w0_v.at[c], ws_hbm.at[i0_v.at[c]], sem))
            handles.append(
                pltpu.async_copy(w1_v.at[c], ws_hbm.at[i1_v.at[c]], sem))
        for h in handles:
            h.wait()

    return k(x, pos0r, pos1r, w0m, w1m)


def _gmlp_body(be_ref, par_ref, bidx_ref, xs_ref, ws_ref, gp_hbm, up_hbm,
               dp_hbm, po_ref, gp_v, up_v, dp_v, sems):
    b = pl.program_id(0)
    e = be_ref[b]
    ec = jnp.minimum(e, E - 1)
    par = par_ref[b]

    def _start(exp, p):
        pltpu.make_async_copy(gp_hbm.at[exp], gp_v.at[p], sems.at[p]).start()
        pltpu.make_async_copy(up_hbm.at[exp], up_v.at[p], sems.at[p]).start()
        pltpu.make_async_copy(dp_hbm.at[exp], dp_v.at[p], sems.at[p]).start()

    def _wait(p):
        pltpu.make_async_copy(gp_hbm.at[0], gp_v.at[p], sems.at[p]).wait()
        pltpu.make_async_copy(up_hbm.at[0], up_v.at[p], sems.at[p]).wait()
        pltpu.make_async_copy(dp_hbm.at[0], dp_v.at[p], sems.at[p]).wait()

    @pl.when(b == 0)
    def _():
        _start(ec, par)

    first = jnp.logical_or(b == 0, e != be_ref[jnp.maximum(b - 1, 0)])

    @pl.when(jnp.logical_and(first, e < E))
    def _():
        _wait(par)

    nxt = be_ref[jnp.minimum(b + 1, NBLK - 1)]

    @pl.when(jnp.logical_and(b + 1 < NBLK,
                             jnp.logical_and(nxt != e, nxt < E)))
    def _():
        _start(nxt, 1 - par)

    @pl.when(e < E)
    def _():
        w = xs_ref[...]
        xlo = lax.bitcast_convert_type(w << 16, jnp.float32)
        xhi = lax.bitcast_convert_type(w & jnp.int32(-65536), jnp.float32)
        xb = jnp.concatenate([xlo, xhi], axis=1)
        hg = _dot_t_bf16(xb, gp_v.at[par][...])
        hu = _dot_t_bf16(xb, up_v.at[par][...])
        h = _silu(hg) * hu
        eo = ws_ref[:, 0:1] * _dot_t_bf16(h, dp_v.at[par][...])
        elo = eo[:, :D // 2].astype(jnp.bfloat16).astype(jnp.float32)
        ehi = eo[:, D // 2:].astype(jnp.bfloat16).astype(jnp.float32)
        po_ref[...] = (((lax.bitcast_convert_type(elo, jnp.int32) >> 16)
                        & jnp.int32(0xFFFF))
                       | (lax.bitcast_convert_type(ehi, jnp.int32)
                          & jnp.int32(-65536)))


def _gmlp(blk_exp, blk_par, blk_idx, xs, ws, gp, up, dp):
    def _bmap(b, be, par, bidx):
        return (bidx[b], 0)

    grid_spec = pltpu.PrefetchScalarGridSpec(
        num_scalar_prefetch=3,
        grid=(NBLK,),
        in_specs=[
            pl.BlockSpec((BM, D // 2), _bmap),
            pl.BlockSpec((BM, 128), _bmap),
            pl.BlockSpec(memory_space=pl.ANY),
            pl.BlockSpec(memory_space=pl.ANY),
            pl.BlockSpec(memory_space=pl.ANY),
        ],
        out_specs=pl.BlockSpec((BM, D // 2), _bmap),
        scratch_shapes=[
            pltpu.VMEM((2, I, D), jnp.float32),
            pltpu.VMEM((2, I, D), jnp.float32),
            pltpu.VMEM((2, D, I), jnp.float32),
            pltpu.SemaphoreType.DMA((2,)),
        ],
    )
    return pl.pallas_call(
        _gmlp_body, grid_spec=grid_spec,
        out_shape=jax.ShapeDtypeStruct((S, D // 2), jnp.int32),
    )(blk_exp, blk_par, blk_idx, xs, ws, gp, up, dp)


CCH = 16
CNCH = T // (NW * CCH)


def _combine(po, pos0r, pos1r, sh):
    mesh = plsc.VectorSubcoreMesh(core_axis_name="c", subcore_axis_name="s")

    @functools.partial(
        pl.kernel, mesh=mesh,
        out_type=jax.ShapeDtypeStruct((T, D), jnp.float32),
        scratch_types=[
            pltpu.VMEM((2, CCH), jnp.int32),
            pltpu.VMEM((2, CCH), jnp.int32),
            pltpu.VMEM((CCH, D), jnp.float32),
            pltpu.VMEM((2, CCH, D // 2), jnp.int32),
            pltpu.VMEM((2, CCH, D // 2), jnp.int32),
            pltpu.SemaphoreType.DMA,
            pltpu.SemaphoreType.DMA,
        ],
    )
    def k(po_hbm, pos0_hbm, pos1_hbm, sh_hbm, out_hbm,
          i0_v, i1_v, acc_v, g0_v, g1_v, sem_a, sem_b):
        wid = lax.axis_index("s") * NC + lax.axis_index("c")
        nv = D // 16
        sems = (sem_a, sem_b)

        def fire(c):
            pr = c % 2
            row = wid * CNCH + c
            pltpu.sync_copy(pos0_hbm.at[row], i0_v.at[pr])
            pltpu.sync_copy(pos1_hbm.at[row], i1_v.at[pr])
            return (pltpu.async_copy(po_hbm.at[i0_v.at[pr]], g0_v.at[pr],
                                     sems[pr]),
                    pltpu.async_copy(po_hbm.at[i1_v.at[pr]], g1_v.at[pr],
                                     sems[pr]))

        pending = fire(0)
        for c in range(CNCH):
            pr = c % 2
            base = (wid * CNCH + c) * CCH
            pltpu.sync_copy(sh_hbm.at[pl.ds(base, CCH)], acc_v)
            h0, h1 = pending
            h0.wait()
            h1.wait()
            if c + 1 < CNCH:
                pending = fire(c + 1)

            def addrow(r, carry):
                for v in range(D // 32):
                    sl = pl.ds(v * 16, 16)
                    w0 = g0_v[pr, r, sl]
                    w1 = g1_v[pr, r, sl]
                    lo = (lax.bitcast_convert_type(w0 << 16, jnp.float32)
                          + lax.bitcast_convert_type(w1 << 16, jnp.float32))
                    hi = (lax.bitcast_convert_type(w0 & jnp.int32(-65536),
                                                   jnp.float32)
                          + lax.bitcast_convert_type(w1 & jnp.int32(-65536),
                                                     jnp.float32))
                    plsc.addupdate(acc_v.at[r, sl], lo)
                    plsc.addupdate(acc_v.at[r, pl.ds(D // 2 + v * 16, 16)],
                                   hi)
                return carry

            lax.fori_loop(0, CCH, addrow, 0)
            pltpu.sync_copy(acc_v, out_hbm.at[pl.ds(base, CCH)])

    return k(po, pos0r, pos1r, sh)


def kernel(hidden_state, gate_w, gate_proj, up_proj, down_proj, shared_gate,
           shared_up, shared_down):
    Bv, Nv, Dv = hidden_state.shape
    x = hidden_state.reshape(Bv * Nv, Dv)
    logits, ids, pos0, pos1, w0m, w1m, be, bpar, bidx, xbf = _router(
        x, gate_w)
    xs, ws = _dispatch(xbf, pos0.reshape(T // CH, CH),
                       pos1.reshape(T // CH, CH), w0m, w1m)
    sh = _shared(x, shared_gate, shared_up, shared_down)
    po = _gmlp(be.reshape(NBLK), bpar.reshape(NBLK), bidx.reshape(NBLK),
               xs, ws, gate_proj, up_proj, down_proj)
    out = _combine(po, pos0.reshape(T // CCH, CCH),
                   pos1.reshape(T // CCH, CCH), sh)
    return out.reshape(Bv, Nv, Dv), logits, ids


# speedup vs baseline: 1.1178x; 1.0299x over previous
import functools

import jax
import jax.numpy as jnp
from jax import lax
from jax.experimental import pallas as pl
from jax.experimental.pallas import tpu as pltpu
from jax.experimental.pallas import tpu_sc as plsc

T, D, I, E, K = 2048, 1024, 512, 8, 2
BM = 256
S = 6144
NBLK = S // BM
NC, NS = 2, 16
NW = NC * NS
CH = 64
NCHUNK = T // (NW * CH)


def _dot_t(a, b):
    return lax.dot_general(a, b, (((1,), (1,)), ((), ())))


def _dot_t_bf16(a, b):
    return lax.dot_general(a.astype(jnp.bfloat16), b.astype(jnp.bfloat16),
                           (((1,), (1,)), ((), ())),
                           preferred_element_type=jnp.float32)


def _silu(x):
    return x / (1.0 + jnp.exp(-x))


def _router_body(x_ref, gate_w_ref,
                 logits_ref, ids_ref, pos0_ref, pos1_ref,
                 w0_ref, w1_ref, be_ref, par_ref, bidx_ref, xbf_ref):
    x = x_ref[...]
    lo = x[:, :D // 2].astype(jnp.bfloat16).astype(jnp.float32)
    hi = x[:, D // 2:].astype(jnp.bfloat16).astype(jnp.float32)
    xbf_ref[...] = (((lax.bitcast_convert_type(lo, jnp.int32) >> 16)
                     & jnp.int32(0xFFFF))
                    | (lax.bitcast_convert_type(hi, jnp.int32)
                       & jnp.int32(-65536)))
    logits = _dot_t(x, gate_w_ref[...])
    logits_ref[...] = logits
    m = jnp.max(logits, axis=1, keepdims=True)
    p = jnp.exp(logits - m)
    probs = p / jnp.sum(p, axis=1, keepdims=True)
    iota_e = lax.broadcasted_iota(jnp.int32, (T, E), 1)
    m1 = jnp.max(probs, axis=1, keepdims=True)
    a1 = jnp.min(jnp.where(probs == m1, iota_e, E), axis=1, keepdims=True)
    probs2 = jnp.where(iota_e == a1, -1.0, probs)
    m2 = jnp.max(probs2, axis=1, keepdims=True)
    a2 = jnp.min(jnp.where(probs2 == m2, iota_e, E), axis=1, keepdims=True)
    s = m1 + m2 + 1e-9
    ids_ref[...] = jnp.concatenate([a1, a2], axis=1)
    w0_ref[...] = jnp.broadcast_to(m1 / s, (T, 128))
    w1_ref[...] = jnp.broadcast_to(m2 / s, (T, 128))

    h1 = (iota_e == a1).astype(jnp.int32)
    h2 = (iota_e == a2).astype(jnp.int32)
    h = h1 + h2
    c = h
    sh = 1
    while sh < T:
        c = c + jnp.concatenate(
            [jnp.zeros((sh, E), jnp.int32), c[:T - sh]], axis=0)
        sh *= 2
    cexcl = c - h
    counts = c[T - 1:T, :]
    pc = ((counts + (BM - 1)) // BM) * BM
    cp = pc
    sh = 1
    while sh < E:
        cp = cp + jnp.concatenate(
            [jnp.zeros((1, sh), jnp.int32), cp[:, :E - sh]], axis=1)
        sh *= 2
    offs = cp - pc
    dest = offs + cexcl
    pos0_ref[...] = jnp.sum(jnp.where(iota_e == a1, dest, 0), axis=1,
                            keepdims=True)
    pos1_ref[...] = jnp.sum(jnp.where(iota_e == a2, dest, 0), axis=1,
                            keepdims=True)
    iota_b = lax.broadcasted_iota(jnp.int32, (NBLK, E), 0) * BM
    be = jnp.sum((iota_b >= jnp.broadcast_to(cp, (NBLK, E)))
                 .astype(jnp.int32), axis=1, keepdims=True)
    be_ref[...] = be
    trans = (be != jnp.concatenate([be[:1], be[:NBLK - 1]], axis=0)
             ).astype(jnp.int32)
    ct = trans
    sh2 = 1
    while sh2 < NBLK:
        ct = ct + jnp.concatenate(
            [jnp.zeros((sh2, 1), jnp.int32), ct[:NBLK - sh2]], axis=0)
        sh2 *= 2
    par_ref[...] = lax.rem(ct, 2)
    lastr = cp[:, E - 1:E] // BM - 1
    iota_r = lax.broadcasted_iota(jnp.int32, (NBLK, 1), 0)
    bidx_ref[...] = jnp.minimum(iota_r, jnp.broadcast_to(lastr, (NBLK, 1)))


def _shared_body(x_ref, sg_ref, su_ref, sd_ref, sh_ref):
    x = x_ref[...]
    hg = _dot_t_bf16(x, sg_ref[...])
    hu = _dot_t_bf16(x, su_ref[...])
    sh_ref[...] = _dot_t_bf16(_silu(hg) * hu, sd_ref[...])


def _shared(x, sg, su, sd):
    return pl.pallas_call(
        _shared_body,
        grid=(T // 256,),
        in_specs=[
            pl.BlockSpec((256, D), lambda i: (i, 0)),
            pl.BlockSpec((I, D), lambda i: (0, 0)),
            pl.BlockSpec((I, D), lambda i: (0, 0)),
            pl.BlockSpec((D, I), lambda i: (0, 0)),
        ],
        out_specs=pl.BlockSpec((256, D), lambda i: (i, 0)),
        out_shape=jax.ShapeDtypeStruct((T, D), jnp.float32),
    )(x, sg, su, sd)


def _router(x, gate_w):
    return pl.pallas_call(
        _router_body,
        out_shape=[
            jax.ShapeDtypeStruct((T, E), jnp.float32),
            jax.ShapeDtypeStruct((T, K), jnp.int32),
            jax.ShapeDtypeStruct((T, 1), jnp.int32),
            jax.ShapeDtypeStruct((T, 1), jnp.int32),
            jax.ShapeDtypeStruct((T, 128), jnp.float32),
            jax.ShapeDtypeStruct((T, 128), jnp.float32),
            jax.ShapeDtypeStruct((NBLK, 1), jnp.int32),
            jax.ShapeDtypeStruct((NBLK, 1), jnp.int32),
            jax.ShapeDtypeStruct((NBLK, 1), jnp.int32),
            jax.ShapeDtypeStruct((T, D // 2), jnp.int32),
        ],
    )(x, gate_w)


def _dispatch(x, pos0r, pos1r, w0m, w1m):
    mesh = plsc.VectorSubcoreMesh(core_axis_name="c", subcore_axis_name="s")

    @functools.partial(
        pl.kernel, mesh=mesh,
        out_type=[jax.ShapeDtypeStruct((S, D // 2), jnp.int32),
                  jax.ShapeDtypeStruct((S, 128), jnp.float32)],
        scratch_types=[
            pltpu.VMEM((NCHUNK, CH), jnp.int32),
            pltpu.VMEM((NCHUNK, CH), jnp.int32),
            pltpu.VMEM((NCHUNK, CH, D // 2), jnp.int32),
            pltpu.VMEM((NCHUNK, CH, 128), jnp.float32),
            pltpu.VMEM((NCHUNK, CH, 128), jnp.float32),
            pltpu.SemaphoreType.DMA,
        ],
    )
    def k(x_hbm, pos0_hbm, pos1_hbm, w0_hbm, w1_hbm, xs_hbm, ws_hbm,
          i0_v, i1_v, xr_v, w0_v, w1_v, sem):
        wid = lax.axis_index("s") * NC + lax.axis_index("c")
        handles = []
        for c in range(NCHUNK):
            row = wid * NCHUNK + c
            base = row * CH
            pltpu.sync_copy(pos0_hbm.at[row], i0_v.at[c])
            pltpu.sync_copy(pos1_hbm.at[row], i1_v.at[c])
            pltpu.sync_copy(x_hbm.at[pl.ds(base, CH)], xr_v.at[c])
            pltpu.sync_copy(w0_hbm.at[pl.ds(base, CH)], w0_v.at[c])
            pltpu.sync_copy(w1_hbm.at[pl.ds(base, CH)], w1_v.at[c])
            handles.append(
                pltpu.async_copy(xr_v.at[c], xs_hbm.at[i0_v.at[c]], sem))
            handles.append(
                pltpu.async_copy(xr_v.at[c], xs_hbm.at[i1_v.at[c]], sem))
            handles.append(
                pltpu.async_copy(w0_v.at[c], ws_hbm.at[i0_v.at[c]], sem))
            handles.append(
                pltpu.async_copy(w1_v.at[c], ws_hbm.at[i1_v.at[c]], sem))
        for h in handles:
            h.wait()

    return k(x, pos0r, pos1r, w0m, w1m)


def _gmlp_body(be_ref, par_ref, bidx_ref, xs_ref, ws_ref, gp_hbm, up_hbm,
               dp_hbm, po_ref, gp_v, up_v, dp_v, sems):
    b = pl.program_id(0)
    e = be_ref[b]
    ec = jnp.minimum(e, E - 1)
    par = par_ref[b]

    def _start(exp, p):
        pltpu.make_async_copy(gp_hbm.at[exp], gp_v.at[p], sems.at[p]).start()
        pltpu.make_async_copy(up_hbm.at[exp], up_v.at[p], sems.at[p]).start()
        pltpu.make_async_copy(dp_hbm.at[exp], dp_v.at[p], sems.at[p]).start()

    def _wait(p):
        pltpu.make_async_copy(gp_hbm.at[0], gp_v.at[p], sems.at[p]).wait()
        pltpu.make_async_copy(up_hbm.at[0], up_v.at[p], sems.at[p]).wait()
        pltpu.make_async_copy(dp_hbm.at[0], dp_v.at[p], sems.at[p]).wait()

    @pl.when(b == 0)
    def _():
        _start(ec, par)

    first = jnp.logical_or(b == 0, e != be_ref[jnp.maximum(b - 1, 0)])

    @pl.when(jnp.logical_and(first, e < E))
    def _():
        _wait(par)

    nxt = be_ref[jnp.minimum(b + 1, NBLK - 1)]

    @pl.when(jnp.logical_and(b + 1 < NBLK,
                             jnp.logical_and(nxt != e, nxt < E)))
    def _():
        _start(nxt, 1 - par)

    @pl.when(e < E)
    def _():
        w = xs_ref[...]
        xlo = lax.bitcast_convert_type(w << 16, jnp.float32)
        xhi = lax.bitcast_convert_type(w & jnp.int32(-65536), jnp.float32)
        xb = jnp.concatenate([xlo, xhi], axis=1)
        hg = _dot_t_bf16(xb, gp_v.at[par][...])
        hu = _dot_t_bf16(xb, up_v.at[par][...])
        h = _silu(hg) * hu
        eo = ws_ref[:, 0:1] * _dot_t_bf16(h, dp_v.at[par][...])
        elo = eo[:, :D // 2].astype(jnp.bfloat16).astype(jnp.float32)
        ehi = eo[:, D // 2:].astype(jnp.bfloat16).astype(jnp.float32)
        po_ref[...] = (((lax.bitcast_convert_type(elo, jnp.int32) >> 16)
                        & jnp.int32(0xFFFF))
                       | (lax.bitcast_convert_type(ehi, jnp.int32)
                          & jnp.int32(-65536)))


def _gmlp(blk_exp, blk_par, blk_idx, xs, ws, gp, up, dp):
    def _bmap(b, be, par, bidx):
        return (bidx[b], 0)

    grid_spec = pltpu.PrefetchScalarGridSpec(
        num_scalar_prefetch=3,
        grid=(NBLK,),
        in_specs=[
            pl.BlockSpec((BM, D // 2), _bmap),
            pl.BlockSpec((BM, 128), _bmap),
            pl.BlockSpec(memory_space=pl.ANY),
            pl.BlockSpec(memory_space=pl.ANY),
            pl.BlockSpec(memory_space=pl.ANY),
        ],
        out_specs=pl.BlockSpec((BM, D // 2), _bmap),
        scratch_shapes=[
            pltpu.VMEM((2, I, D), jnp.float32),
            pltpu.VMEM((2, I, D), jnp.float32),
            pltpu.VMEM((2, D, I), jnp.float32),
            pltpu.SemaphoreType.DMA((2,)),
        ],
    )
    return pl.pallas_call(
        _gmlp_body, grid_spec=grid_spec,
        out_shape=jax.ShapeDtypeStruct((S, D // 2), jnp.int32),
    )(blk_exp, blk_par, blk_idx, xs, ws, gp, up, dp)


CCH = 32
CNCH = T // (NW * CCH)


def _combine(po, pos0r, pos1r, sh):
    mesh = plsc.VectorSubcoreMesh(core_axis_name="c", subcore_axis_name="s")

    @functools.partial(
        pl.kernel, mesh=mesh,
        out_type=jax.ShapeDtypeStruct((T, D), jnp.float32),
        scratch_types=[
            pltpu.VMEM((2, CCH), jnp.int32),
            pltpu.VMEM((2, CCH), jnp.int32),
            pltpu.VMEM((CCH, D), jnp.float32),
            pltpu.VMEM((2, CCH, D // 2), jnp.int32),
            pltpu.VMEM((2, CCH, D // 2), jnp.int32),
            pltpu.SemaphoreType.DMA,
            pltpu.SemaphoreType.DMA,
        ],
    )
    def k(po_hbm, pos0_hbm, pos1_hbm, sh_hbm, out_hbm,
          i0_v, i1_v, acc_v, g0_v, g1_v, sem_a, sem_b):
        wid = lax.axis_index("s") * NC + lax.axis_index("c")
        nv = D // 16
        sems = (sem_a, sem_b)

        def fire(c):
            pr = c % 2
            row = wid * CNCH + c
            pltpu.sync_copy(pos0_hbm.at[row], i0_v.at[pr])
            pltpu.sync_copy(pos1_hbm.at[row], i1_v.at[pr])
            return (pltpu.async_copy(po_hbm.at[i0_v.at[pr]], g0_v.at[pr],
                                     sems[pr]),
                    pltpu.async_copy(po_hbm.at[i1_v.at[pr]], g1_v.at[pr],
                                     sems[pr]))

        pending = fire(0)
        for c in range(CNCH):
            pr = c % 2
            base = (wid * CNCH + c) * CCH
            pltpu.sync_copy(sh_hbm.at[pl.ds(base, CCH)], acc_v)
            h0, h1 = pending
            h0.wait()
            h1.wait()
            if c + 1 < CNCH:
                pending = fire(c + 1)

            def addrow(r, carry):
                for v in range(D // 32):
                    sl = pl.ds(v * 16, 16)
                    w0 = g0_v[pr, r, sl]
                    w1 = g1_v[pr, r, sl]
                    lo = (lax.bitcast_convert_type(w0 << 16, jnp.float32)
                          + lax.bitcast_convert_type(w1 << 16, jnp.float32))
                    hi = (lax.bitcast_convert_type(w0 & jnp.int32(-65536),
                                                   jnp.float32)
                          + lax.bitcast_convert_type(w1 & jnp.int32(-65536),
                                                     jnp.float32))
                    plsc.addupdate(acc_v.at[r, sl], lo)
                    plsc.addupdate(acc_v.at[r, pl.ds(D // 2 + v * 16, 16)],
                                   hi)
                return carry

            lax.fori_loop(0, CCH, addrow, 0)
            pltpu.sync_copy(acc_v, out_hbm.at[pl.ds(base, CCH)])

    return k(po, pos0r, pos1r, sh)


def kernel(hidden_state, gate_w, gate_proj, up_proj, down_proj, shared_gate,
           shared_up, shared_down):
    Bv, Nv, Dv = hidden_state.shape
    x = hidden_state.reshape(Bv * Nv, Dv)
    logits, ids, pos0, pos1, w0m, w1m, be, bpar, bidx, xbf = _router(
        x, gate_w)
    xs, ws = _dispatch(xbf, pos0.reshape(T // CH, CH),
                       pos1.reshape(T // CH, CH), w0m, w1m)
    sh = _shared(x, shared_gate, shared_up, shared_down)
    po = _gmlp(be.reshape(NBLK), bpar.reshape(NBLK), bidx.reshape(NBLK),
               xs, ws, gate_proj, up_proj, down_proj)
    out = _combine(po, pos0.reshape(T // CCH, CCH),
                   pos1.reshape(T // CCH, CCH), sh)
    return out.reshape(Bv, Nv, Dv), logits, ids
